# baseline (speedup 1.0000x reference)
import jax
import jax.numpy as jnp
from jax.experimental import pallas as pl
from jax.experimental.pallas import tpu as pltpu

_VOCAB = 100000
_D = 64
_BT = 256
_E_TILE = 5000
_KA = _VOCAB // _E_TILE
_TILE_V = 16384
_NB = (_VOCAB + _TILE_V - 1) // _TILE_V


def _body(ids_ref, emb_ref, w_ref, o_ref, h_scr):
    i = pl.program_id(0)

    @pl.when(i < _KA)
    def _():
        @pl.when(i == 0)
        def _():
            h_scr[...] = jnp.zeros((_BT, _D), jnp.float32)

        rows = _E_TILE * i + jax.lax.broadcasted_iota(
            jnp.int32, (1, _E_TILE), 1)
        oh = (ids_ref[...] == rows).astype(jnp.bfloat16)
        emb_bf = emb_ref[...].astype(jnp.bfloat16)
        h_scr[...] += jnp.dot(oh, emb_bf,
                              preferred_element_type=jnp.float32)

    @pl.when(i >= _KA)
    def _():
        o_ref[...] = jnp.dot(h_scr[...], w_ref[...],
                             preferred_element_type=jnp.float32)


def kernel(x, emb, W):
    b, t = x.shape
    ids = x.reshape(_BT, 1).astype(jnp.int32)
    logits = pl.pallas_call(
        _body,
        grid=(_KA + _NB,),
        in_specs=[
            pl.BlockSpec((_BT, 1), lambda i: (0, 0)),
            pl.BlockSpec((_E_TILE, _D),
                         lambda i: (jnp.minimum(i, _KA - 1), 0)),
            pl.BlockSpec((_D, _TILE_V),
                         lambda i: (0, jnp.maximum(i - _KA, 0))),
        ],
        out_specs=pl.BlockSpec((_BT, _TILE_V),
                               lambda i: (0, jnp.maximum(i - _KA, 0))),
        out_shape=jax.ShapeDtypeStruct((_BT, _VOCAB), jnp.float32),
        scratch_shapes=[pltpu.VMEM((_BT, _D), jnp.float32)],
        compiler_params=pltpu.CompilerParams(
            dimension_semantics=("arbitrary",)),
    )(ids, emb, W)
    return logits.reshape(b, t, _VOCAB)


# speedup vs baseline: 1.2860x; 1.2860x over previous
import jax
import jax.numpy as jnp
from jax.experimental import pallas as pl
from jax.experimental.pallas import tpu as pltpu

_VOCAB = 100000
_D = 64
_BT = 256
_NM = 2
_MB = _BT // _NM
_TILE_V = 16384
_NV = 6
_V_MAIN = _NV * _TILE_V
_V_TAIL = _VOCAB - _V_MAIN
_NB = _NV + 1


def _body(ids_ref, emb_hbm, w_hbm, wt_ref, o_ref, h_scr, w_scr, gsem, wsem):
    m = pl.program_id(0)
    v = pl.program_id(1)

    @pl.when((m == 0) & (v == 0))
    def _():
        for j in range(_BT):
            pltpu.make_async_copy(
                emb_hbm.at[ids_ref[j]], h_scr.at[j],
                gsem.at[j // _MB]).start()
        for c in range(_NV):
            pltpu.make_async_copy(
                w_hbm.at[:, pl.ds(c * _TILE_V, _TILE_V)],
                w_scr.at[:, pl.ds(c * _TILE_V, _TILE_V)],
                wsem.at[c]).start()

    @pl.when(v == 0)
    def _():
        pltpu.make_async_copy(
            emb_hbm.at[pl.ds(0, _MB)],
            h_scr.at[pl.ds(m * _MB, _MB)], gsem.at[m]).wait()

    @pl.when((m == 0) & (v < _NV))
    def _():
        pltpu.make_async_copy(
            w_hbm.at[:, pl.ds(v * _TILE_V, _TILE_V)],
            w_scr.at[:, pl.ds(v * _TILE_V, _TILE_V)],
            wsem.at[v]).wait()

    mb = pl.multiple_of(m * _MB, _MB)
    vb = pl.multiple_of(v * _TILE_V, _TILE_V)
    h = h_scr[pl.ds(mb, _MB), :]

    @pl.when(v < _NV)
    def _():
        o_ref[...] = jnp.dot(h, w_scr[:, pl.ds(vb, _TILE_V)],
                             preferred_element_type=jnp.float32)

    @pl.when(v == _NV)
    def _():
        o_ref[:, : _V_TAIL] = jnp.dot(h, wt_ref[...],
                                      preferred_element_type=jnp.float32)


def kernel(x, emb, W):
    b, t = x.shape
    ids = x.reshape(_BT).astype(jnp.int32)
    w_tail = W[:, _V_MAIN:]
    grid_spec = pltpu.PrefetchScalarGridSpec(
        num_scalar_prefetch=1,
        grid=(_NM, _NB),
        in_specs=[
            pl.BlockSpec(memory_space=pl.ANY),
            pl.BlockSpec(memory_space=pl.ANY),
            pl.BlockSpec((_D, _V_TAIL), lambda m, v, ids_ref: (0, 0)),
        ],
        out_specs=pl.BlockSpec((_MB, _TILE_V),
                               lambda m, v, ids_ref: (m, v)),
        scratch_shapes=[
            pltpu.VMEM((_BT, _D), jnp.float32),
            pltpu.VMEM((_D, _V_MAIN), jnp.float32),
            pltpu.SemaphoreType.DMA((_NM,)),
            pltpu.SemaphoreType.DMA((_NV,)),
        ],
    )
    logits = pl.pallas_call(
        _body,
        grid_spec=grid_spec,
        out_shape=jax.ShapeDtypeStruct((_BT, _VOCAB), jnp.float32),
        compiler_params=pltpu.CompilerParams(
            dimension_semantics=("arbitrary", "arbitrary"),
            vmem_limit_bytes=60 * 1024 * 1024),
    )(ids, emb, W, w_tail)
    return logits.reshape(b, t, _VOCAB)


# speedup vs baseline: 1.3254x; 1.0306x over previous
import jax
import jax.numpy as jnp
from jax.experimental import pallas as pl
from jax.experimental.pallas import tpu as pltpu

_VOCAB = 100000
_D = 64
_B, _T = 16, 16
_BT = _B * _T
_NM = 2
_MB = _BT // _NM
_MBB = _MB // _T
_TILE_V = 16384
_NV = 6
_V_MAIN = _NV * _TILE_V
_V_TAIL = _VOCAB - _V_MAIN
_TAIL_BLK = 2048
_NB = _NV + 1


def _body(ids_ref, emb_hbm, w_hbm, wt_ref, o_ref, h_scr, w_scr, gsem, wsem):
    m = pl.program_id(0)
    v = pl.program_id(1)

    @pl.when((m == 0) & (v == 0))
    def _():
        for j in range(_BT):
            pltpu.make_async_copy(
                emb_hbm.at[ids_ref[j // _T, j % _T]], h_scr.at[j],
                gsem.at[j // _MB]).start()
        for c in range(_NV):
            pltpu.make_async_copy(
                w_hbm.at[:, pl.ds(c * _TILE_V, _TILE_V)],
                w_scr.at[:, pl.ds(c * _TILE_V, _TILE_V)],
                wsem.at[c]).start()

    @pl.when(v == 0)
    def _():
        pltpu.make_async_copy(
            emb_hbm.at[pl.ds(0, _MB)],
            h_scr.at[pl.ds(m * _MB, _MB)], gsem.at[m]).wait()

    @pl.when((m == 0) & (v < _NV))
    def _():
        pltpu.make_async_copy(
            w_hbm.at[:, pl.ds(v * _TILE_V, _TILE_V)],
            w_scr.at[:, pl.ds(v * _TILE_V, _TILE_V)],
            wsem.at[v]).wait()

    mb = pl.multiple_of(m * _MB, _MB)
    vb = pl.multiple_of(v * _TILE_V, _TILE_V)
    h = h_scr[pl.ds(mb, _MB), :]

    @pl.when(v < _NV)
    def _():
        o_ref[...] = jnp.dot(
            h, w_scr[:, pl.ds(vb, _TILE_V)],
            preferred_element_type=jnp.float32).reshape(_MBB, _T, _TILE_V)

    @pl.when(v == _NV)
    def _():
        o_ref[:, :, :_TAIL_BLK] = jnp.dot(
            h, wt_ref[...],
            preferred_element_type=jnp.float32).reshape(_MBB, _T, _TAIL_BLK)


def kernel(x, emb, W):
    ids = x.astype(jnp.int32)
    grid_spec = pltpu.PrefetchScalarGridSpec(
        num_scalar_prefetch=1,
        grid=(_NM, _NB),
        in_specs=[
            pl.BlockSpec(memory_space=pl.ANY),
            pl.BlockSpec(memory_space=pl.ANY),
            pl.BlockSpec((_D, _TAIL_BLK),
                         lambda m, v, ids_ref: (0, _V_MAIN // _TAIL_BLK)),
        ],
        out_specs=pl.BlockSpec((_MBB, _T, _TILE_V),
                               lambda m, v, ids_ref: (m, 0, v)),
        scratch_shapes=[
            pltpu.VMEM((_BT, _D), jnp.float32),
            pltpu.VMEM((_D, _V_MAIN), jnp.float32),
            pltpu.SemaphoreType.DMA((_NM,)),
            pltpu.SemaphoreType.DMA((_NV,)),
        ],
    )
    return pl.pallas_call(
        _body,
        grid_spec=grid_spec,
        out_shape=jax.ShapeDtypeStruct((_B, _T, _VOCAB), jnp.float32),
        compiler_params=pltpu.CompilerParams(
            dimension_semantics=("arbitrary", "arbitrary"),
            vmem_limit_bytes=60 * 1024 * 1024),
    )(ids, emb, W, W)
